# Initial kernel scaffold; baseline (speedup 1.0000x reference)
import jax
import jax.numpy as jnp
from jax.experimental import pallas as pl


def kernel(x, edge_index, attention, conv_z_W, conv_z_b, lin_z_W, lin_z_b, conv_r_W, conv_r_b, lin_r_W, lin_r_b, conv_h_W, conv_h_b, lin_h_W, lin_h_b, out_W, out_b):
    raise NotImplementedError("write your pallas kernel here")


# speedup vs baseline: 50.3518x; 50.3518x over previous
import functools

import jax
import jax.numpy as jnp
from jax import lax
from jax.experimental import pallas as pl
from jax.experimental.pallas import tpu as pltpu
from jax.experimental.pallas import tpu_sc as plsc

NA = 10000
EDG = 320000
FI = 128
FO = 32
NPER = 12
NCHUNK = 6
NP = 10112
RPT_A = NP // 16
KB = 128
NBLK = 160
GB = 8
ECP = 16 * NBLK * KB
KA = 128
NBLKA = 80
EDP = 32 * NBLKA * KA
BN = 400

_mesh = plsc.VectorSubcoreMesh(core_axis_name="c", subcore_axis_name="s")


@functools.partial(
    pl.kernel,
    out_type=jax.ShapeDtypeStruct((2 * NP, FI), jnp.float32),
    mesh=_mesh,
    scratch_types=[
        pltpu.VMEM((NBLKA, KA), jnp.int32),
        pltpu.VMEM((KA, FI), jnp.float32),
        pltpu.VMEM_SHARED((NP, FI), jnp.float32),
    ],
)
def _deg_kernel(didx_hbm, init_hbm, out_hbm, idx_v, ones_v, acc_sh):
    c = lax.axis_index("c")
    s = lax.axis_index("s")
    gw = s * 2 + c
    pltpu.sync_copy(didx_hbm.at[pl.ds(gw * NBLKA, NBLKA)], idx_v)
    pltpu.sync_copy(init_hbm.at[pl.ds(0, KA)], ones_v)
    r0 = s * RPT_A
    pltpu.sync_copy(init_hbm.at[pl.ds(c * NP + r0, RPT_A)], acc_sh.at[pl.ds(r0, RPT_A)])
    plsc.subcore_barrier()

    def body(j, carry):
        pltpu.sync_copy(ones_v, acc_sh.at[idx_v.at[j]], add=True)
        return carry

    lax.fori_loop(0, NBLKA, body, 0)
    plsc.subcore_barrier()
    pltpu.sync_copy(acc_sh.at[pl.ds(r0, RPT_A)], out_hbm.at[pl.ds(c * NP + r0, RPT_A)])


@functools.partial(
    pl.kernel,
    out_type=jax.ShapeDtypeStruct((NCHUNK * NP, FI), jnp.float32),
    mesh=_mesh,
    scratch_types=[
        pltpu.VMEM((GB, KB), jnp.int32),
        pltpu.VMEM((GB, KB), jnp.int32),
        pltpu.VMEM((KB, FI), jnp.float32),
        pltpu.VMEM_SHARED((NP, FI), jnp.float32),
        pltpu.SemaphoreType.DMA,
    ],
)
def _agg_kernel(y_hbm, sidx_hbm, didx_hbm, out_hbm, sidx_v, didx_v, gbuf, acc_sh, sem):
    c = lax.axis_index("c")
    s = lax.axis_index("s")
    r0 = s * RPT_A
    for i in range(3):
        chunk = c * 3 + i
        pltpu.sync_copy(y_hbm.at[pl.ds(chunk * NP + r0, RPT_A)], acc_sh.at[pl.ds(r0, RPT_A)])
        plsc.subcore_barrier()

        def body(jo, carry):
            pltpu.sync_copy(
                sidx_hbm.at[pl.ds((chunk * 16 + s) * NBLK + jo * GB, GB)], sidx_v)
            pltpu.sync_copy(didx_hbm.at[pl.ds(s * NBLK + jo * GB, GB)], didx_v)
            for g in range(GB):
                pltpu.async_copy(y_hbm.at[sidx_v.at[g]], gbuf, sem).wait()
                pltpu.sync_copy(gbuf, acc_sh.at[didx_v.at[g]], add=True)
            return carry

        lax.fori_loop(0, NBLK // GB, body, 0)
        plsc.subcore_barrier()
        pltpu.sync_copy(acc_sh.at[pl.ds(r0, RPT_A)], out_hbm.at[pl.ds(chunk * NP + r0, RPT_A)])
        plsc.subcore_barrier()


def _mm_body(x_ref, czw_ref, lzw_ref, chw_ref, lhw_ref, deg_ref, out_ref):
    wz = jnp.dot(czw_ref[...], lzw_ref[...], preferred_element_type=jnp.float32)
    wh = jnp.dot(chw_ref[...], lhw_ref[...], preferred_element_type=jnp.float32)
    w = jnp.concatenate([wz, wh], axis=1)
    dinv = lax.rsqrt(deg_ref[0, :, 0:1] + deg_ref[1, :, 0:1])
    y0 = jnp.dot(x_ref[0], w, preferred_element_type=jnp.float32)
    y1 = jnp.dot(x_ref[1], w, preferred_element_type=jnp.float32)
    out_ref[0] = jnp.concatenate([y0, y1], axis=1) * dinv


_mm_call = pl.pallas_call(
    _mm_body,
    grid=(NCHUNK, NA // BN),
    in_specs=[
        pl.BlockSpec((2, BN, FI), lambda ci, i: (ci, i, 0)),
        pl.BlockSpec((FI, FO), lambda ci, i: (0, 0)),
        pl.BlockSpec((FO, FO), lambda ci, i: (0, 0)),
        pl.BlockSpec((FI, FO), lambda ci, i: (0, 0)),
        pl.BlockSpec((FO, FO), lambda ci, i: (0, 0)),
        pl.BlockSpec((2, BN, FI), lambda ci, i: (0, i, 0)),
    ],
    out_specs=pl.BlockSpec((1, BN, FI), lambda ci, i: (ci, i, 0)),
    out_shape=jax.ShapeDtypeStruct((NCHUNK, NP, FI), jnp.float32),
)


def _fin_body(agg_ref, deg_ref, att_ref, czb_ref, lzw_ref, lzb_ref, chb_ref,
              lhw_ref, lhb_ref, wout_ref, bout_ref, out_ref):
    probs = jax.nn.softmax(att_ref[...], axis=1)
    bz = jnp.dot(czb_ref[...], lzw_ref[...], preferred_element_type=jnp.float32) + lzb_ref[...]
    bh = jnp.dot(chb_ref[...], lhw_ref[...], preferred_element_type=jnp.float32) + lhb_ref[...]
    dinv = lax.rsqrt(deg_ref[0, :, 0:1] + deg_ref[1, :, 0:1])
    hacc = jnp.zeros((BN, FO), dtype=jnp.float32)
    for p in range(NPER):
        cch = p // 2
        off = 64 * (p % 2)
        a = agg_ref[cch, :, off:off + FO] * dinv + bz
        b = agg_ref[cch, :, off + FO:off + 2 * FO] * dinv + bh
        hp = (1.0 - jax.nn.sigmoid(a)) * jnp.tanh(b)
        hacc = hacc + probs[0, p] * hp
    out_ref[...] = (jnp.dot(jnp.maximum(hacc, 0.0), wout_ref[...],
                            preferred_element_type=jnp.float32) + bout_ref[...])


_fin_call = pl.pallas_call(
    _fin_body,
    grid=(NA // BN,),
    in_specs=[
        pl.BlockSpec((NCHUNK, BN, FI), lambda i: (0, i, 0)),
        pl.BlockSpec((2, BN, FI), lambda i: (0, i, 0)),
        pl.BlockSpec((1, NPER), lambda i: (0, 0)),
        pl.BlockSpec((1, FO), lambda i: (0, 0)),
        pl.BlockSpec((FO, FO), lambda i: (0, 0)),
        pl.BlockSpec((1, FO), lambda i: (0, 0)),
        pl.BlockSpec((1, FO), lambda i: (0, 0)),
        pl.BlockSpec((FO, FO), lambda i: (0, 0)),
        pl.BlockSpec((1, FO), lambda i: (0, 0)),
        pl.BlockSpec((FO, NPER), lambda i: (0, 0)),
        pl.BlockSpec((1, NPER), lambda i: (0, 0)),
    ],
    out_specs=pl.BlockSpec((BN, NPER), lambda i: (i, 0)),
    out_shape=jax.ShapeDtypeStruct((NA, NPER), jnp.float32),
)


def kernel(x, edge_index, attention, conv_z_W, conv_z_b, lin_z_W, lin_z_b,
           conv_r_W, conv_r_b, lin_r_W, lin_r_b, conv_h_W, conv_h_b,
           lin_h_W, lin_h_b, out_W, out_b):
    src = edge_index[0]
    dst = edge_index[1]

    pad = ECP - EDG
    ar = jnp.arange(pad, dtype=jnp.int32)
    src_p = jnp.concatenate([src, (ar * 97) % NA]).reshape(16 * NBLK, KB)
    dst_p = jnp.concatenate([dst, NA + (ar % (NP - NA))]).reshape(16 * NBLK, KB)
    sidx = (src_p[None] + (jnp.arange(NCHUNK, dtype=jnp.int32) * NP)[:, None, None])
    sidx = sidx.reshape(NCHUNK * 16 * NBLK, KB)

    ard = jnp.arange(EDP - EDG, dtype=jnp.int32)
    dstd = jnp.concatenate([dst, NA + (ard % (NP - NA))]).reshape(32 * NBLKA, KA)
    init = jnp.concatenate([jnp.ones((NP, FI), jnp.float32),
                            jnp.zeros((NP, FI), jnp.float32)])

    deg2 = _deg_kernel(dstd, init).reshape(2, NP, FI)
    x_t = jnp.transpose(x, (2, 0, 1))
    y = _mm_call(x_t, conv_z_W, lin_z_W[:FO], conv_h_W, lin_h_W[:FO], deg2)
    agg = _agg_kernel(y.reshape(NCHUNK * NP, FI), sidx, dst_p)
    out = _fin_call(agg.reshape(NCHUNK, NP, FI), deg2, attention.reshape(1, NPER),
                    conv_z_b.reshape(1, FO), lin_z_W[:FO], lin_z_b.reshape(1, FO),
                    conv_h_b.reshape(1, FO), lin_h_W[:FO], lin_h_b.reshape(1, FO),
                    out_W, out_b.reshape(1, NPER))
    return out


# speedup vs baseline: 61.4458x; 1.2203x over previous
import functools

import jax
import jax.numpy as jnp
from jax import lax
from jax.experimental import pallas as pl
from jax.experimental.pallas import tpu as pltpu
from jax.experimental.pallas import tpu_sc as plsc

NA = 10000
EDG = 320000
FI = 128
FO = 32
NPER = 12
NCHUNK = 6
NP = 10112
RPT_A = NP // 16
KB = 128
NBLK = 160
GB = 8
ECP = 16 * NBLK * KB
KA = 128
NBLKA = 80
EDP = 32 * NBLKA * KA
BN = 400

_mesh = plsc.VectorSubcoreMesh(core_axis_name="c", subcore_axis_name="s")


@functools.partial(
    pl.kernel,
    out_type=jax.ShapeDtypeStruct((2 * NP, FI), jnp.float32),
    mesh=_mesh,
    scratch_types=[
        pltpu.VMEM((NBLKA, KA), jnp.int32),
        pltpu.VMEM((KA, FI), jnp.float32),
        pltpu.VMEM_SHARED((NP, FI), jnp.float32),
    ],
)
def _deg_kernel(didx_hbm, init_hbm, out_hbm, idx_v, ones_v, acc_sh):
    c = lax.axis_index("c")
    s = lax.axis_index("s")
    gw = s * 2 + c
    pltpu.sync_copy(didx_hbm.at[pl.ds(gw * NBLKA, NBLKA)], idx_v)
    pltpu.sync_copy(init_hbm.at[pl.ds(0, KA)], ones_v)
    r0 = s * RPT_A
    pltpu.sync_copy(init_hbm.at[pl.ds(c * NP + r0, RPT_A)], acc_sh.at[pl.ds(r0, RPT_A)])
    plsc.subcore_barrier()

    def body(j, carry):
        pltpu.sync_copy(ones_v, acc_sh.at[idx_v.at[j]], add=True)
        return carry

    lax.fori_loop(0, NBLKA, body, 0)
    plsc.subcore_barrier()
    pltpu.sync_copy(acc_sh.at[pl.ds(r0, RPT_A)], out_hbm.at[pl.ds(c * NP + r0, RPT_A)])


@functools.partial(
    pl.kernel,
    out_type=jax.ShapeDtypeStruct((NCHUNK * NP, FI), jnp.float32),
    mesh=_mesh,
    scratch_types=[
        pltpu.VMEM((GB, KB), jnp.int32),
        pltpu.VMEM((GB, KB), jnp.int32),
        pltpu.VMEM((2, KB, FI), jnp.float32),
        pltpu.VMEM_SHARED((NP, FI), jnp.float32),
        pltpu.SemaphoreType.DMA,
        pltpu.SemaphoreType.DMA,
    ],
)
def _agg_kernel(y_hbm, sidx_hbm, didx_hbm, out_hbm, sidx_v, didx_v, gbuf, acc_sh,
                sem_a, sem_b):
    c = lax.axis_index("c")
    s = lax.axis_index("s")
    r0 = s * RPT_A
    sems = (sem_a, sem_b)
    for i in range(3):
        chunk = c * 3 + i
        pltpu.sync_copy(y_hbm.at[pl.ds(chunk * NP + r0, RPT_A)], acc_sh.at[pl.ds(r0, RPT_A)])
        plsc.subcore_barrier()

        def body(jo, carry):
            pltpu.sync_copy(
                sidx_hbm.at[pl.ds((chunk * 16 + s) * NBLK + jo * GB, GB)], sidx_v)
            pltpu.sync_copy(didx_hbm.at[pl.ds(s * NBLK + jo * GB, GB)], didx_v)
            pltpu.async_copy(y_hbm.at[sidx_v.at[0]], gbuf.at[0], sems[0])
            for g in range(GB):
                cur = g % 2
                pltpu.make_async_copy(
                    y_hbm.at[sidx_v.at[g]], gbuf.at[cur], sems[cur]).wait()
                if g + 1 < GB:
                    pltpu.async_copy(
                        y_hbm.at[sidx_v.at[g + 1]], gbuf.at[1 - cur], sems[1 - cur])
                pltpu.sync_copy(gbuf.at[cur], acc_sh.at[didx_v.at[g]], add=True)
            return carry

        lax.fori_loop(0, NBLK // GB, body, 0)
        plsc.subcore_barrier()
        pltpu.sync_copy(acc_sh.at[pl.ds(r0, RPT_A)], out_hbm.at[pl.ds(chunk * NP + r0, RPT_A)])
        plsc.subcore_barrier()


def _mm_body(x_ref, czw_ref, lzw_ref, chw_ref, lhw_ref, deg_ref, out_ref):
    wz = jnp.dot(czw_ref[...], lzw_ref[...], preferred_element_type=jnp.float32)
    wh = jnp.dot(chw_ref[...], lhw_ref[...], preferred_element_type=jnp.float32)
    w = jnp.concatenate([wz, wh], axis=1)
    dinv = lax.rsqrt(deg_ref[0, :, 0:1] + deg_ref[1, :, 0:1])
    y0 = jnp.dot(x_ref[0], w, preferred_element_type=jnp.float32)
    y1 = jnp.dot(x_ref[1], w, preferred_element_type=jnp.float32)
    out_ref[0] = jnp.concatenate([y0, y1], axis=1) * dinv


_mm_call = pl.pallas_call(
    _mm_body,
    grid=(NCHUNK, NA // BN),
    in_specs=[
        pl.BlockSpec((2, BN, FI), lambda ci, i: (ci, i, 0)),
        pl.BlockSpec((FI, FO), lambda ci, i: (0, 0)),
        pl.BlockSpec((FO, FO), lambda ci, i: (0, 0)),
        pl.BlockSpec((FI, FO), lambda ci, i: (0, 0)),
        pl.BlockSpec((FO, FO), lambda ci, i: (0, 0)),
        pl.BlockSpec((2, BN, FI), lambda ci, i: (0, i, 0)),
    ],
    out_specs=pl.BlockSpec((1, BN, FI), lambda ci, i: (ci, i, 0)),
    out_shape=jax.ShapeDtypeStruct((NCHUNK, NP, FI), jnp.float32),
)


def _fin_body(agg_ref, deg_ref, att_ref, czb_ref, lzw_ref, lzb_ref, chb_ref,
              lhw_ref, lhb_ref, wout_ref, bout_ref, out_ref):
    probs = jax.nn.softmax(att_ref[...], axis=1)
    bz = jnp.dot(czb_ref[...], lzw_ref[...], preferred_element_type=jnp.float32) + lzb_ref[...]
    bh = jnp.dot(chb_ref[...], lhw_ref[...], preferred_element_type=jnp.float32) + lhb_ref[...]
    dinv = lax.rsqrt(deg_ref[0, :, 0:1] + deg_ref[1, :, 0:1])
    hacc = jnp.zeros((BN, FO), dtype=jnp.float32)
    for p in range(NPER):
        cch = p // 2
        off = 64 * (p % 2)
        a = agg_ref[cch, :, off:off + FO] * dinv + bz
        b = agg_ref[cch, :, off + FO:off + 2 * FO] * dinv + bh
        hp = (1.0 - jax.nn.sigmoid(a)) * jnp.tanh(b)
        hacc = hacc + probs[0, p] * hp
    out_ref[...] = (jnp.dot(jnp.maximum(hacc, 0.0), wout_ref[...],
                            preferred_element_type=jnp.float32) + bout_ref[...])


_fin_call = pl.pallas_call(
    _fin_body,
    grid=(NA // BN,),
    in_specs=[
        pl.BlockSpec((NCHUNK, BN, FI), lambda i: (0, i, 0)),
        pl.BlockSpec((2, BN, FI), lambda i: (0, i, 0)),
        pl.BlockSpec((1, NPER), lambda i: (0, 0)),
        pl.BlockSpec((1, FO), lambda i: (0, 0)),
        pl.BlockSpec((FO, FO), lambda i: (0, 0)),
        pl.BlockSpec((1, FO), lambda i: (0, 0)),
        pl.BlockSpec((1, FO), lambda i: (0, 0)),
        pl.BlockSpec((FO, FO), lambda i: (0, 0)),
        pl.BlockSpec((1, FO), lambda i: (0, 0)),
        pl.BlockSpec((FO, NPER), lambda i: (0, 0)),
        pl.BlockSpec((1, NPER), lambda i: (0, 0)),
    ],
    out_specs=pl.BlockSpec((BN, NPER), lambda i: (i, 0)),
    out_shape=jax.ShapeDtypeStruct((NA, NPER), jnp.float32),
)


def kernel(x, edge_index, attention, conv_z_W, conv_z_b, lin_z_W, lin_z_b,
           conv_r_W, conv_r_b, lin_r_W, lin_r_b, conv_h_W, conv_h_b,
           lin_h_W, lin_h_b, out_W, out_b):
    src = edge_index[0]
    dst = edge_index[1]

    pad = ECP - EDG
    ar = jnp.arange(pad, dtype=jnp.int32)
    src_p = jnp.concatenate([src, (ar * 97) % NA]).reshape(16 * NBLK, KB)
    dst_p = jnp.concatenate([dst, NA + (ar % (NP - NA))]).reshape(16 * NBLK, KB)
    sidx = (src_p[None] + (jnp.arange(NCHUNK, dtype=jnp.int32) * NP)[:, None, None])
    sidx = sidx.reshape(NCHUNK * 16 * NBLK, KB)

    ard = jnp.arange(EDP - EDG, dtype=jnp.int32)
    dstd = jnp.concatenate([dst, NA + (ard % (NP - NA))]).reshape(32 * NBLKA, KA)
    init = jnp.concatenate([jnp.ones((NP, FI), jnp.float32),
                            jnp.zeros((NP, FI), jnp.float32)])

    deg2 = _deg_kernel(dstd, init).reshape(2, NP, FI)
    x_t = jnp.transpose(x, (2, 0, 1))
    y = _mm_call(x_t, conv_z_W, lin_z_W[:FO], conv_h_W, lin_h_W[:FO], deg2)
    agg = _agg_kernel(y.reshape(NCHUNK * NP, FI), sidx, dst_p)
    out = _fin_call(agg.reshape(NCHUNK, NP, FI), deg2, attention.reshape(1, NPER),
                    conv_z_b.reshape(1, FO), lin_z_W[:FO], lin_z_b.reshape(1, FO),
                    conv_h_b.reshape(1, FO), lin_h_W[:FO], lin_h_b.reshape(1, FO),
                    out_W, out_b.reshape(1, NPER))
    return out


# speedup vs baseline: 63.9758x; 1.0412x over previous
import functools

import jax
import jax.numpy as jnp
from jax import lax
from jax.experimental import pallas as pl
from jax.experimental.pallas import tpu as pltpu
from jax.experimental.pallas import tpu_sc as plsc

NA = 10000
EDG = 320000
FI = 128
FO = 32
NPER = 12
NCHUNK = 6
NP = 10112
RPT_A = NP // 16
KB = 128
NBLK = 160
GB = 16
ECP = 16 * NBLK * KB
KA = 128
NBLKA = 80
EDP = 32 * NBLKA * KA
BN = 400

_mesh = plsc.VectorSubcoreMesh(core_axis_name="c", subcore_axis_name="s")


@functools.partial(
    pl.kernel,
    out_type=jax.ShapeDtypeStruct((2 * NP, FI), jnp.float32),
    mesh=_mesh,
    scratch_types=[
        pltpu.VMEM((NBLKA, KA), jnp.int32),
        pltpu.VMEM((KA, FI), jnp.float32),
        pltpu.VMEM_SHARED((NP, FI), jnp.float32),
    ],
)
def _deg_kernel(didx_hbm, init_hbm, out_hbm, idx_v, ones_v, acc_sh):
    c = lax.axis_index("c")
    s = lax.axis_index("s")
    gw = s * 2 + c
    pltpu.sync_copy(didx_hbm.at[pl.ds(gw * NBLKA, NBLKA)], idx_v)
    pltpu.sync_copy(init_hbm.at[pl.ds(0, KA)], ones_v)
    r0 = s * RPT_A
    pltpu.sync_copy(init_hbm.at[pl.ds(c * NP + r0, RPT_A)], acc_sh.at[pl.ds(r0, RPT_A)])
    plsc.subcore_barrier()

    def body(j, carry):
        pltpu.sync_copy(ones_v, acc_sh.at[idx_v.at[j]], add=True)
        return carry

    lax.fori_loop(0, NBLKA, body, 0)
    plsc.subcore_barrier()
    pltpu.sync_copy(acc_sh.at[pl.ds(r0, RPT_A)], out_hbm.at[pl.ds(c * NP + r0, RPT_A)])


@functools.partial(
    pl.kernel,
    out_type=jax.ShapeDtypeStruct((NCHUNK * NP, FI), jnp.float32),
    mesh=_mesh,
    scratch_types=[
        pltpu.VMEM((GB, KB), jnp.int32),
        pltpu.VMEM((GB, KB), jnp.int32),
        pltpu.VMEM((2, KB, FI), jnp.float32),
        pltpu.VMEM_SHARED((NP, FI), jnp.float32),
        pltpu.SemaphoreType.DMA,
        pltpu.SemaphoreType.DMA,
    ],
)
def _agg_kernel(y_hbm, sidx_hbm, didx_hbm, out_hbm, sidx_v, didx_v, gbuf, acc_sh,
                sem_a, sem_b):
    c = lax.axis_index("c")
    s = lax.axis_index("s")
    r0 = s * RPT_A
    sems = (sem_a, sem_b)
    for i in range(3):
        chunk = c * 3 + i
        pltpu.sync_copy(y_hbm.at[pl.ds(chunk * NP + r0, RPT_A)], acc_sh.at[pl.ds(r0, RPT_A)])
        plsc.subcore_barrier()

        def body(jo, carry):
            pltpu.sync_copy(
                sidx_hbm.at[pl.ds((chunk * 16 + s) * NBLK + jo * GB, GB)], sidx_v)
            pltpu.sync_copy(didx_hbm.at[pl.ds(s * NBLK + jo * GB, GB)], didx_v)
            pltpu.async_copy(y_hbm.at[sidx_v.at[0]], gbuf.at[0], sems[0])
            for g in range(GB):
                cur = g % 2
                pltpu.make_async_copy(
                    y_hbm.at[sidx_v.at[g]], gbuf.at[cur], sems[cur]).wait()
                if g + 1 < GB:
                    pltpu.async_copy(
                        y_hbm.at[sidx_v.at[g + 1]], gbuf.at[1 - cur], sems[1 - cur])
                pltpu.sync_copy(gbuf.at[cur], acc_sh.at[didx_v.at[g]], add=True)
            return carry

        lax.fori_loop(0, NBLK // GB, body, 0)
        plsc.subcore_barrier()
        pltpu.sync_copy(acc_sh.at[pl.ds(r0, RPT_A)], out_hbm.at[pl.ds(chunk * NP + r0, RPT_A)])
        plsc.subcore_barrier()


def _mm_body(x_ref, czw_ref, lzw_ref, chw_ref, lhw_ref, deg_ref, out_ref):
    wz = jnp.dot(czw_ref[...], lzw_ref[...], preferred_element_type=jnp.float32)
    wh = jnp.dot(chw_ref[...], lhw_ref[...], preferred_element_type=jnp.float32)
    w = jnp.concatenate([wz, wh], axis=1)
    dinv = lax.rsqrt(deg_ref[0, :, 0:1] + deg_ref[1, :, 0:1])
    y0 = jnp.dot(x_ref[0], w, preferred_element_type=jnp.float32)
    y1 = jnp.dot(x_ref[1], w, preferred_element_type=jnp.float32)
    out_ref[0] = jnp.concatenate([y0, y1], axis=1) * dinv


_mm_call = pl.pallas_call(
    _mm_body,
    grid=(NCHUNK, NA // BN),
    in_specs=[
        pl.BlockSpec((2, BN, FI), lambda ci, i: (ci, i, 0)),
        pl.BlockSpec((FI, FO), lambda ci, i: (0, 0)),
        pl.BlockSpec((FO, FO), lambda ci, i: (0, 0)),
        pl.BlockSpec((FI, FO), lambda ci, i: (0, 0)),
        pl.BlockSpec((FO, FO), lambda ci, i: (0, 0)),
        pl.BlockSpec((2, BN, FI), lambda ci, i: (0, i, 0)),
    ],
    out_specs=pl.BlockSpec((1, BN, FI), lambda ci, i: (ci, i, 0)),
    out_shape=jax.ShapeDtypeStruct((NCHUNK, NP, FI), jnp.float32),
)


def _fin_body(agg_ref, deg_ref, att_ref, czb_ref, lzw_ref, lzb_ref, chb_ref,
              lhw_ref, lhb_ref, wout_ref, bout_ref, out_ref):
    probs = jax.nn.softmax(att_ref[...], axis=1)
    bz = jnp.dot(czb_ref[...], lzw_ref[...], preferred_element_type=jnp.float32) + lzb_ref[...]
    bh = jnp.dot(chb_ref[...], lhw_ref[...], preferred_element_type=jnp.float32) + lhb_ref[...]
    dinv = lax.rsqrt(deg_ref[0, :, 0:1] + deg_ref[1, :, 0:1])
    hacc = jnp.zeros((BN, FO), dtype=jnp.float32)
    for p in range(NPER):
        cch = p // 2
        off = 64 * (p % 2)
        a = agg_ref[cch, :, off:off + FO] * dinv + bz
        b = agg_ref[cch, :, off + FO:off + 2 * FO] * dinv + bh
        hp = (1.0 - jax.nn.sigmoid(a)) * jnp.tanh(b)
        hacc = hacc + probs[0, p] * hp
    out_ref[...] = (jnp.dot(jnp.maximum(hacc, 0.0), wout_ref[...],
                            preferred_element_type=jnp.float32) + bout_ref[...])


_fin_call = pl.pallas_call(
    _fin_body,
    grid=(NA // BN,),
    in_specs=[
        pl.BlockSpec((NCHUNK, BN, FI), lambda i: (0, i, 0)),
        pl.BlockSpec((2, BN, FI), lambda i: (0, i, 0)),
        pl.BlockSpec((1, NPER), lambda i: (0, 0)),
        pl.BlockSpec((1, FO), lambda i: (0, 0)),
        pl.BlockSpec((FO, FO), lambda i: (0, 0)),
        pl.BlockSpec((1, FO), lambda i: (0, 0)),
        pl.BlockSpec((1, FO), lambda i: (0, 0)),
        pl.BlockSpec((FO, FO), lambda i: (0, 0)),
        pl.BlockSpec((1, FO), lambda i: (0, 0)),
        pl.BlockSpec((FO, NPER), lambda i: (0, 0)),
        pl.BlockSpec((1, NPER), lambda i: (0, 0)),
    ],
    out_specs=pl.BlockSpec((BN, NPER), lambda i: (i, 0)),
    out_shape=jax.ShapeDtypeStruct((NA, NPER), jnp.float32),
)


def kernel(x, edge_index, attention, conv_z_W, conv_z_b, lin_z_W, lin_z_b,
           conv_r_W, conv_r_b, lin_r_W, lin_r_b, conv_h_W, conv_h_b,
           lin_h_W, lin_h_b, out_W, out_b):
    src = edge_index[0]
    dst = edge_index[1]

    pad = ECP - EDG
    ar = jnp.arange(pad, dtype=jnp.int32)
    src_p = jnp.concatenate([src, (ar * 97) % NA]).reshape(16 * NBLK, KB)
    dst_p = jnp.concatenate([dst, NA + (ar % (NP - NA))]).reshape(16 * NBLK, KB)
    sidx = (src_p[None] + (jnp.arange(NCHUNK, dtype=jnp.int32) * NP)[:, None, None])
    sidx = sidx.reshape(NCHUNK * 16 * NBLK, KB)

    ard = jnp.arange(EDP - EDG, dtype=jnp.int32)
    dstd = jnp.concatenate([dst, NA + (ard % (NP - NA))]).reshape(32 * NBLKA, KA)
    init = jnp.concatenate([jnp.ones((NP, FI), jnp.float32),
                            jnp.zeros((NP, FI), jnp.float32)])

    deg2 = _deg_kernel(dstd, init).reshape(2, NP, FI)
    x_t = jnp.transpose(x, (2, 0, 1))
    y = _mm_call(x_t, conv_z_W, lin_z_W[:FO], conv_h_W, lin_h_W[:FO], deg2)
    agg = _agg_kernel(y.reshape(NCHUNK * NP, FI), sidx, dst_p)
    out = _fin_call(agg.reshape(NCHUNK, NP, FI), deg2, attention.reshape(1, NPER),
                    conv_z_b.reshape(1, FO), lin_z_W[:FO], lin_z_b.reshape(1, FO),
                    conv_h_b.reshape(1, FO), lin_h_W[:FO], lin_h_b.reshape(1, FO),
                    out_W, out_b.reshape(1, NPER))
    return out


# speedup vs baseline: 65.0678x; 1.0171x over previous
import functools

import jax
import jax.numpy as jnp
from jax import lax
from jax.experimental import pallas as pl
from jax.experimental.pallas import tpu as pltpu
from jax.experimental.pallas import tpu_sc as plsc

NA = 10000
EDG = 320000
FI = 128
FO = 32
NPER = 12
NCHUNK = 6
NP = 10112
RPT_A = NP // 16
KB = 128
NBLK = 160
GB = 16
ECP = 16 * NBLK * KB
KA = 128
NBLKA = 80
EDP = 32 * NBLKA * KA
BN = 400

_mesh = plsc.VectorSubcoreMesh(core_axis_name="c", subcore_axis_name="s")


@functools.partial(
    pl.kernel,
    out_type=jax.ShapeDtypeStruct((2 * NP, FI), jnp.float32),
    mesh=_mesh,
    scratch_types=[
        pltpu.VMEM((NBLKA, KA), jnp.int32),
        pltpu.VMEM((KA, FI), jnp.float32),
        pltpu.VMEM_SHARED((NP, FI), jnp.float32),
    ],
)
def _deg_kernel(didx_hbm, init_hbm, out_hbm, idx_v, ones_v, acc_sh):
    c = lax.axis_index("c")
    s = lax.axis_index("s")
    gw = s * 2 + c
    pltpu.sync_copy(didx_hbm.at[pl.ds(gw * NBLKA, NBLKA)], idx_v)
    pltpu.sync_copy(init_hbm.at[pl.ds(0, KA)], ones_v)
    r0 = s * RPT_A
    pltpu.sync_copy(init_hbm.at[pl.ds(c * NP + r0, RPT_A)], acc_sh.at[pl.ds(r0, RPT_A)])
    plsc.subcore_barrier()

    def body(j, carry):
        pltpu.sync_copy(ones_v, acc_sh.at[idx_v.at[j]], add=True)
        return carry

    lax.fori_loop(0, NBLKA, body, 0)
    plsc.subcore_barrier()
    pltpu.sync_copy(acc_sh.at[pl.ds(r0, RPT_A)], out_hbm.at[pl.ds(c * NP + r0, RPT_A)])


@functools.partial(
    pl.kernel,
    out_type=jax.ShapeDtypeStruct((NCHUNK * NP, FI), jnp.float32),
    mesh=_mesh,
    scratch_types=[
        pltpu.VMEM((2 * GB, KB // 2), jnp.int32),
        pltpu.VMEM((GB, KB), jnp.int32),
        pltpu.VMEM((2, KB, FI), jnp.float32),
        pltpu.VMEM_SHARED((NP, FI), jnp.float32),
        pltpu.SemaphoreType.DMA,
        pltpu.SemaphoreType.DMA,
        pltpu.SemaphoreType.DMA,
        pltpu.SemaphoreType.DMA,
    ],
)
def _agg_kernel(y_hbm, sidx_hbm, didx_hbm, out_hbm, sidx_v, didx_v, gbuf, acc_sh,
                sem_a, sem_b, sem_c, sem_d):
    c = lax.axis_index("c")
    s = lax.axis_index("s")
    r0 = s * RPT_A
    sems = ((sem_a, sem_b), (sem_c, sem_d))

    def _start_gather(y_hbm, sidx_v, gbuf, blk, buf):
        for h in range(2):
            pltpu.async_copy(y_hbm.at[sidx_v.at[2 * blk + h]],
                             gbuf.at[buf, pl.ds(h * (KB // 2), KB // 2)],
                             sems[buf][h])

    def _wait_gather(y_hbm, sidx_v, gbuf, blk, buf):
        for h in range(2):
            pltpu.make_async_copy(y_hbm.at[sidx_v.at[2 * blk + h]],
                                  gbuf.at[buf, pl.ds(h * (KB // 2), KB // 2)],
                                  sems[buf][h]).wait()
    for i in range(3):
        chunk = c * 3 + i
        pltpu.sync_copy(y_hbm.at[pl.ds(chunk * NP + r0, RPT_A)], acc_sh.at[pl.ds(r0, RPT_A)])
        plsc.subcore_barrier()

        def body(jo, carry):
            pltpu.sync_copy(
                sidx_hbm.at[pl.ds(((chunk * 16 + s) * NBLK + jo * GB) * 2, 2 * GB)],
                sidx_v)
            pltpu.sync_copy(didx_hbm.at[pl.ds(s * NBLK + jo * GB, GB)], didx_v)
            _start_gather(y_hbm, sidx_v, gbuf, 0, 0)
            for g in range(GB):
                cur = g % 2
                _wait_gather(y_hbm, sidx_v, gbuf, g, cur)
                if g + 1 < GB:
                    _start_gather(y_hbm, sidx_v, gbuf, g + 1, 1 - cur)
                pltpu.sync_copy(gbuf.at[cur], acc_sh.at[didx_v.at[g]], add=True)
            return carry

        lax.fori_loop(0, NBLK // GB, body, 0)
        plsc.subcore_barrier()
        pltpu.sync_copy(acc_sh.at[pl.ds(r0, RPT_A)], out_hbm.at[pl.ds(chunk * NP + r0, RPT_A)])
        plsc.subcore_barrier()


def _mm_body(x_ref, czw_ref, lzw_ref, chw_ref, lhw_ref, deg_ref, out_ref):
    wz = jnp.dot(czw_ref[...], lzw_ref[...], preferred_element_type=jnp.float32)
    wh = jnp.dot(chw_ref[...], lhw_ref[...], preferred_element_type=jnp.float32)
    w = jnp.concatenate([wz, wh], axis=1)
    dinv = lax.rsqrt(deg_ref[0, :, 0:1] + deg_ref[1, :, 0:1])
    y0 = jnp.dot(x_ref[0], w, preferred_element_type=jnp.float32)
    y1 = jnp.dot(x_ref[1], w, preferred_element_type=jnp.float32)
    out_ref[0] = jnp.concatenate([y0, y1], axis=1) * dinv


_mm_call = pl.pallas_call(
    _mm_body,
    grid=(NCHUNK, NA // BN),
    in_specs=[
        pl.BlockSpec((2, BN, FI), lambda ci, i: (ci, i, 0)),
        pl.BlockSpec((FI, FO), lambda ci, i: (0, 0)),
        pl.BlockSpec((FO, FO), lambda ci, i: (0, 0)),
        pl.BlockSpec((FI, FO), lambda ci, i: (0, 0)),
        pl.BlockSpec((FO, FO), lambda ci, i: (0, 0)),
        pl.BlockSpec((2, BN, FI), lambda ci, i: (0, i, 0)),
    ],
    out_specs=pl.BlockSpec((1, BN, FI), lambda ci, i: (ci, i, 0)),
    out_shape=jax.ShapeDtypeStruct((NCHUNK, NP, FI), jnp.float32),
)


def _fin_body(agg_ref, deg_ref, att_ref, czb_ref, lzw_ref, lzb_ref, chb_ref,
              lhw_ref, lhb_ref, wout_ref, bout_ref, out_ref):
    probs = jax.nn.softmax(att_ref[...], axis=1)
    bz = jnp.dot(czb_ref[...], lzw_ref[...], preferred_element_type=jnp.float32) + lzb_ref[...]
    bh = jnp.dot(chb_ref[...], lhw_ref[...], preferred_element_type=jnp.float32) + lhb_ref[...]
    dinv = lax.rsqrt(deg_ref[0, :, 0:1] + deg_ref[1, :, 0:1])
    hacc = jnp.zeros((BN, FO), dtype=jnp.float32)
    for p in range(NPER):
        cch = p // 2
        off = 64 * (p % 2)
        a = agg_ref[cch, :, off:off + FO] * dinv + bz
        b = agg_ref[cch, :, off + FO:off + 2 * FO] * dinv + bh
        hp = (1.0 - jax.nn.sigmoid(a)) * jnp.tanh(b)
        hacc = hacc + probs[0, p] * hp
    out_ref[...] = (jnp.dot(jnp.maximum(hacc, 0.0), wout_ref[...],
                            preferred_element_type=jnp.float32) + bout_ref[...])


_fin_call = pl.pallas_call(
    _fin_body,
    grid=(NA // BN,),
    in_specs=[
        pl.BlockSpec((NCHUNK, BN, FI), lambda i: (0, i, 0)),
        pl.BlockSpec((2, BN, FI), lambda i: (0, i, 0)),
        pl.BlockSpec((1, NPER), lambda i: (0, 0)),
        pl.BlockSpec((1, FO), lambda i: (0, 0)),
        pl.BlockSpec((FO, FO), lambda i: (0, 0)),
        pl.BlockSpec((1, FO), lambda i: (0, 0)),
        pl.BlockSpec((1, FO), lambda i: (0, 0)),
        pl.BlockSpec((FO, FO), lambda i: (0, 0)),
        pl.BlockSpec((1, FO), lambda i: (0, 0)),
        pl.BlockSpec((FO, NPER), lambda i: (0, 0)),
        pl.BlockSpec((1, NPER), lambda i: (0, 0)),
    ],
    out_specs=pl.BlockSpec((BN, NPER), lambda i: (i, 0)),
    out_shape=jax.ShapeDtypeStruct((NA, NPER), jnp.float32),
)


def kernel(x, edge_index, attention, conv_z_W, conv_z_b, lin_z_W, lin_z_b,
           conv_r_W, conv_r_b, lin_r_W, lin_r_b, conv_h_W, conv_h_b,
           lin_h_W, lin_h_b, out_W, out_b):
    src = edge_index[0]
    dst = edge_index[1]

    pad = ECP - EDG
    ar = jnp.arange(pad, dtype=jnp.int32)
    src_p = jnp.concatenate([src, (ar * 97) % NA]).reshape(16 * NBLK, KB)
    dst_p = jnp.concatenate([dst, NA + (ar % (NP - NA))]).reshape(16 * NBLK, KB)
    sidx = (src_p[None] + (jnp.arange(NCHUNK, dtype=jnp.int32) * NP)[:, None, None])
    sidx = sidx.reshape(NCHUNK * 16 * NBLK * 2, KB // 2)

    ard = jnp.arange(EDP - EDG, dtype=jnp.int32)
    dstd = jnp.concatenate([dst, NA + (ard % (NP - NA))]).reshape(32 * NBLKA, KA)
    init = jnp.concatenate([jnp.ones((NP, FI), jnp.float32),
                            jnp.zeros((NP, FI), jnp.float32)])

    deg2 = _deg_kernel(dstd, init).reshape(2, NP, FI)
    x_t = jnp.transpose(x, (2, 0, 1))
    y = _mm_call(x_t, conv_z_W, lin_z_W[:FO], conv_h_W, lin_h_W[:FO], deg2)
    agg = _agg_kernel(y.reshape(NCHUNK * NP, FI), sidx, dst_p)
    out = _fin_call(agg.reshape(NCHUNK, NP, FI), deg2, attention.reshape(1, NPER),
                    conv_z_b.reshape(1, FO), lin_z_W[:FO], lin_z_b.reshape(1, FO),
                    conv_h_b.reshape(1, FO), lin_h_W[:FO], lin_h_b.reshape(1, FO),
                    out_W, out_b.reshape(1, NPER))
    return out


# speedup vs baseline: 65.0863x; 1.0003x over previous
import functools

import jax
import jax.numpy as jnp
from jax import lax
from jax.experimental import pallas as pl
from jax.experimental.pallas import tpu as pltpu
from jax.experimental.pallas import tpu_sc as plsc

NA = 10000
EDG = 320000
FI = 128
FO = 32
NPER = 12
NCHUNK = 6
NP = 10112
RPT_A = NP // 16
KB = 128
NBLK = 160
GB = 16
ECP = 16 * NBLK * KB
KA = 128
NBLKA = 80
EDP = 32 * NBLKA * KA
BN = 400
DW = 128

_mesh = plsc.VectorSubcoreMesh(core_axis_name="c", subcore_axis_name="s")


@functools.partial(
    pl.kernel,
    out_type=jax.ShapeDtypeStruct((2 * NP, DW), jnp.float32),
    mesh=_mesh,
    scratch_types=[
        pltpu.VMEM((NBLKA, KA), jnp.int32),
        pltpu.VMEM((KA, DW), jnp.float32),
        pltpu.VMEM_SHARED((NP, DW), jnp.float32),
    ],
)
def _deg_kernel(didx_hbm, init_hbm, out_hbm, idx_v, ones_v, acc_sh):
    c = lax.axis_index("c")
    s = lax.axis_index("s")
    gw = s * 2 + c
    pltpu.sync_copy(didx_hbm.at[pl.ds(gw * NBLKA, NBLKA)], idx_v)
    pltpu.sync_copy(init_hbm.at[pl.ds(0, KA)], ones_v)
    r0 = s * RPT_A
    pltpu.sync_copy(init_hbm.at[pl.ds(c * NP + r0, RPT_A)], acc_sh.at[pl.ds(r0, RPT_A)])
    plsc.subcore_barrier()

    def body(j, carry):
        pltpu.sync_copy(ones_v, acc_sh.at[idx_v.at[j]], add=True)
        return carry

    lax.fori_loop(0, NBLKA, body, 0)
    plsc.subcore_barrier()
    pltpu.sync_copy(acc_sh.at[pl.ds(r0, RPT_A)], out_hbm.at[pl.ds(c * NP + r0, RPT_A)])


@functools.partial(
    pl.kernel,
    out_type=jax.ShapeDtypeStruct((NCHUNK * NP, FI), jnp.float32),
    mesh=_mesh,
    scratch_types=[
        pltpu.VMEM((2 * GB, KB // 2), jnp.int32),
        pltpu.VMEM((GB, KB), jnp.int32),
        pltpu.VMEM((2, KB, FI), jnp.float32),
        pltpu.VMEM_SHARED((NP, FI), jnp.float32),
        pltpu.SemaphoreType.DMA,
        pltpu.SemaphoreType.DMA,
        pltpu.SemaphoreType.DMA,
        pltpu.SemaphoreType.DMA,
    ],
)
def _agg_kernel(y_hbm, sidx_hbm, didx_hbm, out_hbm, sidx_v, didx_v, gbuf, acc_sh,
                sem_a, sem_b, sem_c, sem_d):
    c = lax.axis_index("c")
    s = lax.axis_index("s")
    r0 = s * RPT_A
    sems = ((sem_a, sem_b), (sem_c, sem_d))

    def _start_gather(y_hbm, sidx_v, gbuf, blk, buf):
        for h in range(2):
            pltpu.async_copy(y_hbm.at[sidx_v.at[2 * blk + h]],
                             gbuf.at[buf, pl.ds(h * (KB // 2), KB // 2)],
                             sems[buf][h])

    def _wait_gather(y_hbm, sidx_v, gbuf, blk, buf):
        for h in range(2):
            pltpu.make_async_copy(y_hbm.at[sidx_v.at[2 * blk + h]],
                                  gbuf.at[buf, pl.ds(h * (KB // 2), KB // 2)],
                                  sems[buf][h]).wait()
    for i in range(3):
        chunk = c * 3 + i
        pltpu.sync_copy(y_hbm.at[pl.ds(chunk * NP + r0, RPT_A)], acc_sh.at[pl.ds(r0, RPT_A)])
        plsc.subcore_barrier()

        def body(jo, carry):
            pltpu.sync_copy(
                sidx_hbm.at[pl.ds(((chunk * 16 + s) * NBLK + jo * GB) * 2, 2 * GB)],
                sidx_v)
            pltpu.sync_copy(didx_hbm.at[pl.ds(s * NBLK + jo * GB, GB)], didx_v)
            _start_gather(y_hbm, sidx_v, gbuf, 0, 0)
            for g in range(GB):
                cur = g % 2
                _wait_gather(y_hbm, sidx_v, gbuf, g, cur)
                if g + 1 < GB:
                    _start_gather(y_hbm, sidx_v, gbuf, g + 1, 1 - cur)
                pltpu.sync_copy(gbuf.at[cur], acc_sh.at[didx_v.at[g]], add=True)
            return carry

        lax.fori_loop(0, NBLK // GB, body, 0)
        plsc.subcore_barrier()
        pltpu.sync_copy(acc_sh.at[pl.ds(r0, RPT_A)], out_hbm.at[pl.ds(chunk * NP + r0, RPT_A)])
        plsc.subcore_barrier()


def _mm_body(x_ref, czw_ref, lzw_ref, chw_ref, lhw_ref, deg_ref, out_ref):
    wz = jnp.dot(czw_ref[...], lzw_ref[...], preferred_element_type=jnp.float32)
    wh = jnp.dot(chw_ref[...], lhw_ref[...], preferred_element_type=jnp.float32)
    w = jnp.concatenate([wz, wh], axis=1)
    dinv = lax.rsqrt(deg_ref[0, :, 0:1] + deg_ref[1, :, 0:1])
    y0 = jnp.dot(x_ref[0], w, preferred_element_type=jnp.float32)
    y1 = jnp.dot(x_ref[1], w, preferred_element_type=jnp.float32)
    out_ref[0] = jnp.concatenate([y0, y1], axis=1) * dinv


_mm_call = pl.pallas_call(
    _mm_body,
    grid=(NCHUNK, NA // BN),
    in_specs=[
        pl.BlockSpec((2, BN, FI), lambda ci, i: (ci, i, 0)),
        pl.BlockSpec((FI, FO), lambda ci, i: (0, 0)),
        pl.BlockSpec((FO, FO), lambda ci, i: (0, 0)),
        pl.BlockSpec((FI, FO), lambda ci, i: (0, 0)),
        pl.BlockSpec((FO, FO), lambda ci, i: (0, 0)),
        pl.BlockSpec((2, BN, DW), lambda ci, i: (0, i, 0)),
    ],
    out_specs=pl.BlockSpec((1, BN, FI), lambda ci, i: (ci, i, 0)),
    out_shape=jax.ShapeDtypeStruct((NCHUNK, NP, FI), jnp.float32),
)


def _fin_body(agg_ref, deg_ref, att_ref, czb_ref, lzw_ref, lzb_ref, chb_ref,
              lhw_ref, lhb_ref, wout_ref, bout_ref, out_ref):
    probs = jax.nn.softmax(att_ref[...], axis=1)
    bz = jnp.dot(czb_ref[...], lzw_ref[...], preferred_element_type=jnp.float32) + lzb_ref[...]
    bh = jnp.dot(chb_ref[...], lhw_ref[...], preferred_element_type=jnp.float32) + lhb_ref[...]
    dinv = lax.rsqrt(deg_ref[0, :, 0:1] + deg_ref[1, :, 0:1])
    hacc = jnp.zeros((BN, FO), dtype=jnp.float32)
    for p in range(NPER):
        cch = p // 2
        off = 64 * (p % 2)
        a = agg_ref[cch, :, off:off + FO] * dinv + bz
        b = agg_ref[cch, :, off + FO:off + 2 * FO] * dinv + bh
        hp = (1.0 - jax.nn.sigmoid(a)) * jnp.tanh(b)
        hacc = hacc + probs[0, p] * hp
    out_ref[...] = (jnp.dot(jnp.maximum(hacc, 0.0), wout_ref[...],
                            preferred_element_type=jnp.float32) + bout_ref[...])


_fin_call = pl.pallas_call(
    _fin_body,
    grid=(NA // BN,),
    in_specs=[
        pl.BlockSpec((NCHUNK, BN, FI), lambda i: (0, i, 0)),
        pl.BlockSpec((2, BN, DW), lambda i: (0, i, 0)),
        pl.BlockSpec((1, NPER), lambda i: (0, 0)),
        pl.BlockSpec((1, FO), lambda i: (0, 0)),
        pl.BlockSpec((FO, FO), lambda i: (0, 0)),
        pl.BlockSpec((1, FO), lambda i: (0, 0)),
        pl.BlockSpec((1, FO), lambda i: (0, 0)),
        pl.BlockSpec((FO, FO), lambda i: (0, 0)),
        pl.BlockSpec((1, FO), lambda i: (0, 0)),
        pl.BlockSpec((FO, NPER), lambda i: (0, 0)),
        pl.BlockSpec((1, NPER), lambda i: (0, 0)),
    ],
    out_specs=pl.BlockSpec((BN, NPER), lambda i: (i, 0)),
    out_shape=jax.ShapeDtypeStruct((NA, NPER), jnp.float32),
)


def kernel(x, edge_index, attention, conv_z_W, conv_z_b, lin_z_W, lin_z_b,
           conv_r_W, conv_r_b, lin_r_W, lin_r_b, conv_h_W, conv_h_b,
           lin_h_W, lin_h_b, out_W, out_b):
    src = edge_index[0]
    dst = edge_index[1]

    pad = ECP - EDG
    ar = jnp.arange(pad, dtype=jnp.int32)
    src_p = jnp.concatenate([src, (ar * 97) % NA]).reshape(16 * NBLK, KB)
    dst_p = jnp.concatenate([dst, NA + (ar % (NP - NA))]).reshape(16 * NBLK, KB)
    sidx = (src_p[None] + (jnp.arange(NCHUNK, dtype=jnp.int32) * NP)[:, None, None])
    sidx = sidx.reshape(NCHUNK * 16 * NBLK * 2, KB // 2)

    ard = jnp.arange(EDP - EDG, dtype=jnp.int32)
    dstd = jnp.concatenate([dst, NA + (ard % (NP - NA))]).reshape(32 * NBLKA, KA)
    init = jnp.concatenate([jnp.ones((NP, DW), jnp.float32),
                            jnp.zeros((NP, DW), jnp.float32)])

    deg2 = _deg_kernel(dstd, init).reshape(2, NP, DW)
    x_t = jnp.transpose(x, (2, 0, 1))
    y = _mm_call(x_t, conv_z_W, lin_z_W[:FO], conv_h_W, lin_h_W[:FO], deg2)
    agg = _agg_kernel(y.reshape(NCHUNK * NP, FI), sidx, dst_p)
    out = _fin_call(agg.reshape(NCHUNK, NP, FI), deg2, attention.reshape(1, NPER),
                    conv_z_b.reshape(1, FO), lin_z_W[:FO], lin_z_b.reshape(1, FO),
                    conv_h_b.reshape(1, FO), lin_h_W[:FO], lin_h_b.reshape(1, FO),
                    out_W, out_b.reshape(1, NPER))
    return out


# speedup vs baseline: 66.6573x; 1.0241x over previous
import functools

import jax
import jax.numpy as jnp
from jax import lax
from jax.experimental import pallas as pl
from jax.experimental.pallas import tpu as pltpu
from jax.experimental.pallas import tpu_sc as plsc

NA = 10000
EDG = 320000
FI = 128
FO = 32
NPER = 12
NCHUNK = 6
NP = 10112
RPT_A = NP // 16
KB = 128
NBLK = 160
GB = 32
ECP = 16 * NBLK * KB
KA = 128
NBLKA = 80
EDP = 32 * NBLKA * KA
BN = 400
DW = 128

_mesh = plsc.VectorSubcoreMesh(core_axis_name="c", subcore_axis_name="s")


@functools.partial(
    pl.kernel,
    out_type=jax.ShapeDtypeStruct((2 * NP, DW), jnp.float32),
    mesh=_mesh,
    scratch_types=[
        pltpu.VMEM((NBLKA, KA), jnp.int32),
        pltpu.VMEM((KA, DW), jnp.float32),
        pltpu.VMEM_SHARED((NP, DW), jnp.float32),
    ],
)
def _deg_kernel(didx_hbm, init_hbm, out_hbm, idx_v, ones_v, acc_sh):
    c = lax.axis_index("c")
    s = lax.axis_index("s")
    gw = s * 2 + c
    pltpu.sync_copy(didx_hbm.at[pl.ds(gw * NBLKA, NBLKA)], idx_v)
    pltpu.sync_copy(init_hbm.at[pl.ds(0, KA)], ones_v)
    r0 = s * RPT_A
    pltpu.sync_copy(init_hbm.at[pl.ds(c * NP + r0, RPT_A)], acc_sh.at[pl.ds(r0, RPT_A)])
    plsc.subcore_barrier()

    def body(j, carry):
        pltpu.sync_copy(ones_v, acc_sh.at[idx_v.at[j]], add=True)
        return carry

    lax.fori_loop(0, NBLKA, body, 0)
    plsc.subcore_barrier()
    pltpu.sync_copy(acc_sh.at[pl.ds(r0, RPT_A)], out_hbm.at[pl.ds(c * NP + r0, RPT_A)])


@functools.partial(
    pl.kernel,
    out_type=jax.ShapeDtypeStruct((NCHUNK * NP, FI), jnp.float32),
    mesh=_mesh,
    scratch_types=[
        pltpu.VMEM((2 * GB, KB // 2), jnp.int32),
        pltpu.VMEM((GB, KB), jnp.int32),
        pltpu.VMEM((2, KB, FI), jnp.float32),
        pltpu.VMEM_SHARED((NP, FI), jnp.float32),
        pltpu.SemaphoreType.DMA,
        pltpu.SemaphoreType.DMA,
        pltpu.SemaphoreType.DMA,
        pltpu.SemaphoreType.DMA,
    ],
)
def _agg_kernel(y_hbm, sidx_hbm, didx_hbm, out_hbm, sidx_v, didx_v, gbuf, acc_sh,
                sem_a, sem_b, sem_c, sem_d):
    c = lax.axis_index("c")
    s = lax.axis_index("s")
    r0 = s * RPT_A
    sems = ((sem_a, sem_b), (sem_c, sem_d))

    def _start_gather(y_hbm, sidx_v, gbuf, blk, buf):
        for h in range(2):
            pltpu.async_copy(y_hbm.at[sidx_v.at[2 * blk + h]],
                             gbuf.at[buf, pl.ds(h * (KB // 2), KB // 2)],
                             sems[buf][h])

    def _wait_gather(y_hbm, sidx_v, gbuf, blk, buf):
        for h in range(2):
            pltpu.make_async_copy(y_hbm.at[sidx_v.at[2 * blk + h]],
                                  gbuf.at[buf, pl.ds(h * (KB // 2), KB // 2)],
                                  sems[buf][h]).wait()
    for i in range(3):
        chunk = c * 3 + i
        pltpu.sync_copy(y_hbm.at[pl.ds(chunk * NP + r0, RPT_A)], acc_sh.at[pl.ds(r0, RPT_A)])
        plsc.subcore_barrier()

        def body(jo, carry):
            pltpu.sync_copy(
                sidx_hbm.at[pl.ds(((chunk * 16 + s) * NBLK + jo * GB) * 2, 2 * GB)],
                sidx_v)
            pltpu.sync_copy(didx_hbm.at[pl.ds(s * NBLK + jo * GB, GB)], didx_v)
            _start_gather(y_hbm, sidx_v, gbuf, 0, 0)
            for g in range(GB):
                cur = g % 2
                _wait_gather(y_hbm, sidx_v, gbuf, g, cur)
                if g + 1 < GB:
                    _start_gather(y_hbm, sidx_v, gbuf, g + 1, 1 - cur)
                pltpu.sync_copy(gbuf.at[cur], acc_sh.at[didx_v.at[g]], add=True)
            return carry

        lax.fori_loop(0, NBLK // GB, body, 0)
        plsc.subcore_barrier()
        pltpu.sync_copy(acc_sh.at[pl.ds(r0, RPT_A)], out_hbm.at[pl.ds(chunk * NP + r0, RPT_A)])
        plsc.subcore_barrier()


def _mm_body(x_ref, czw_ref, lzw_ref, chw_ref, lhw_ref, deg_ref, out_ref):
    wz = jnp.dot(czw_ref[...], lzw_ref[...], preferred_element_type=jnp.float32)
    wh = jnp.dot(chw_ref[...], lhw_ref[...], preferred_element_type=jnp.float32)
    w = jnp.concatenate([wz, wh], axis=1)
    dinv = lax.rsqrt(deg_ref[0, :, 0:1] + deg_ref[1, :, 0:1])
    y0 = jnp.dot(x_ref[0], w, preferred_element_type=jnp.float32)
    y1 = jnp.dot(x_ref[1], w, preferred_element_type=jnp.float32)
    out_ref[0] = jnp.concatenate([y0, y1], axis=1) * dinv


_mm_call = pl.pallas_call(
    _mm_body,
    grid=(NCHUNK, NA // BN),
    in_specs=[
        pl.BlockSpec((2, BN, FI), lambda ci, i: (ci, i, 0)),
        pl.BlockSpec((FI, FO), lambda ci, i: (0, 0)),
        pl.BlockSpec((FO, FO), lambda ci, i: (0, 0)),
        pl.BlockSpec((FI, FO), lambda ci, i: (0, 0)),
        pl.BlockSpec((FO, FO), lambda ci, i: (0, 0)),
        pl.BlockSpec((2, BN, DW), lambda ci, i: (0, i, 0)),
    ],
    out_specs=pl.BlockSpec((1, BN, FI), lambda ci, i: (ci, i, 0)),
    out_shape=jax.ShapeDtypeStruct((NCHUNK, NP, FI), jnp.float32),
)


def _fin_body(agg_ref, deg_ref, att_ref, czb_ref, lzw_ref, lzb_ref, chb_ref,
              lhw_ref, lhb_ref, wout_ref, bout_ref, out_ref):
    probs = jax.nn.softmax(att_ref[...], axis=1)
    bz = jnp.dot(czb_ref[...], lzw_ref[...], preferred_element_type=jnp.float32) + lzb_ref[...]
    bh = jnp.dot(chb_ref[...], lhw_ref[...], preferred_element_type=jnp.float32) + lhb_ref[...]
    dinv = lax.rsqrt(deg_ref[0, :, 0:1] + deg_ref[1, :, 0:1])
    hacc = jnp.zeros((BN, FO), dtype=jnp.float32)
    for p in range(NPER):
        cch = p // 2
        off = 64 * (p % 2)
        a = agg_ref[cch, :, off:off + FO] * dinv + bz
        b = agg_ref[cch, :, off + FO:off + 2 * FO] * dinv + bh
        hp = (1.0 - jax.nn.sigmoid(a)) * jnp.tanh(b)
        hacc = hacc + probs[0, p] * hp
    out_ref[...] = (jnp.dot(jnp.maximum(hacc, 0.0), wout_ref[...],
                            preferred_element_type=jnp.float32) + bout_ref[...])


_fin_call = pl.pallas_call(
    _fin_body,
    grid=(NA // BN,),
    in_specs=[
        pl.BlockSpec((NCHUNK, BN, FI), lambda i: (0, i, 0)),
        pl.BlockSpec((2, BN, DW), lambda i: (0, i, 0)),
        pl.BlockSpec((1, NPER), lambda i: (0, 0)),
        pl.BlockSpec((1, FO), lambda i: (0, 0)),
        pl.BlockSpec((FO, FO), lambda i: (0, 0)),
        pl.BlockSpec((1, FO), lambda i: (0, 0)),
        pl.BlockSpec((1, FO), lambda i: (0, 0)),
        pl.BlockSpec((FO, FO), lambda i: (0, 0)),
        pl.BlockSpec((1, FO), lambda i: (0, 0)),
        pl.BlockSpec((FO, NPER), lambda i: (0, 0)),
        pl.BlockSpec((1, NPER), lambda i: (0, 0)),
    ],
    out_specs=pl.BlockSpec((BN, NPER), lambda i: (i, 0)),
    out_shape=jax.ShapeDtypeStruct((NA, NPER), jnp.float32),
)


def kernel(x, edge_index, attention, conv_z_W, conv_z_b, lin_z_W, lin_z_b,
           conv_r_W, conv_r_b, lin_r_W, lin_r_b, conv_h_W, conv_h_b,
           lin_h_W, lin_h_b, out_W, out_b):
    src = edge_index[0]
    dst = edge_index[1]

    pad = ECP - EDG
    ar = jnp.arange(pad, dtype=jnp.int32)
    src_p = jnp.concatenate([src, (ar * 97) % NA]).reshape(16 * NBLK, KB)
    dst_p = jnp.concatenate([dst, NA + (ar % (NP - NA))]).reshape(16 * NBLK, KB)
    sidx = (src_p[None] + (jnp.arange(NCHUNK, dtype=jnp.int32) * NP)[:, None, None])
    sidx = sidx.reshape(NCHUNK * 16 * NBLK * 2, KB // 2)

    ard = jnp.arange(EDP - EDG, dtype=jnp.int32)
    dstd = jnp.concatenate([dst, NA + (ard % (NP - NA))]).reshape(32 * NBLKA, KA)
    init = jnp.concatenate([jnp.ones((NP, DW), jnp.float32),
                            jnp.zeros((NP, DW), jnp.float32)])

    deg2 = _deg_kernel(dstd, init).reshape(2, NP, DW)
    x_t = jnp.transpose(x, (2, 0, 1))
    y = _mm_call(x_t, conv_z_W, lin_z_W[:FO], conv_h_W, lin_h_W[:FO], deg2)
    agg = _agg_kernel(y.reshape(NCHUNK * NP, FI), sidx, dst_p)
    out = _fin_call(agg.reshape(NCHUNK, NP, FI), deg2, attention.reshape(1, NPER),
                    conv_z_b.reshape(1, FO), lin_z_W[:FO], lin_z_b.reshape(1, FO),
                    conv_h_b.reshape(1, FO), lin_h_W[:FO], lin_h_b.reshape(1, FO),
                    out_W, out_b.reshape(1, NPER))
    return out


# speedup vs baseline: 66.9825x; 1.0049x over previous
import functools

import jax
import jax.numpy as jnp
from jax import lax
from jax.experimental import pallas as pl
from jax.experimental.pallas import tpu as pltpu
from jax.experimental.pallas import tpu_sc as plsc

NA = 10000
EDG = 320000
FI = 128
FO = 32
NPER = 12
NCHUNK = 6
NP = 10112
RPT_A = NP // 16
KB = 128
NBLK = 160
GB = 40
ECP = 16 * NBLK * KB
KA = 128
NBLKA = 80
EDP = 32 * NBLKA * KA
BN = 400
DW = 128

_mesh = plsc.VectorSubcoreMesh(core_axis_name="c", subcore_axis_name="s")


@functools.partial(
    pl.kernel,
    out_type=jax.ShapeDtypeStruct((2 * NP, DW), jnp.float32),
    mesh=_mesh,
    scratch_types=[
        pltpu.VMEM((NBLKA, KA), jnp.int32),
        pltpu.VMEM((KA, DW), jnp.float32),
        pltpu.VMEM_SHARED((NP, DW), jnp.float32),
    ],
)
def _deg_kernel(didx_hbm, init_hbm, out_hbm, idx_v, ones_v, acc_sh):
    c = lax.axis_index("c")
    s = lax.axis_index("s")
    gw = s * 2 + c
    pltpu.sync_copy(didx_hbm.at[pl.ds(gw * NBLKA, NBLKA)], idx_v)
    pltpu.sync_copy(init_hbm.at[pl.ds(0, KA)], ones_v)
    r0 = s * RPT_A
    pltpu.sync_copy(init_hbm.at[pl.ds(c * NP + r0, RPT_A)], acc_sh.at[pl.ds(r0, RPT_A)])
    plsc.subcore_barrier()

    def body(j, carry):
        pltpu.sync_copy(ones_v, acc_sh.at[idx_v.at[j]], add=True)
        return carry

    lax.fori_loop(0, NBLKA, body, 0)
    plsc.subcore_barrier()
    pltpu.sync_copy(acc_sh.at[pl.ds(r0, RPT_A)], out_hbm.at[pl.ds(c * NP + r0, RPT_A)])


@functools.partial(
    pl.kernel,
    out_type=jax.ShapeDtypeStruct((NCHUNK * NP, FI), jnp.float32),
    mesh=_mesh,
    scratch_types=[
        pltpu.VMEM((2 * GB, KB // 2), jnp.int32),
        pltpu.VMEM((GB, KB), jnp.int32),
        pltpu.VMEM((2, KB, FI), jnp.float32),
        pltpu.VMEM_SHARED((NP, FI), jnp.float32),
        pltpu.SemaphoreType.DMA,
        pltpu.SemaphoreType.DMA,
        pltpu.SemaphoreType.DMA,
        pltpu.SemaphoreType.DMA,
    ],
)
def _agg_kernel(y_hbm, sidx_hbm, didx_hbm, out_hbm, sidx_v, didx_v, gbuf, acc_sh,
                sem_a, sem_b, sem_c, sem_d):
    c = lax.axis_index("c")
    s = lax.axis_index("s")
    r0 = s * RPT_A
    sems = ((sem_a, sem_b), (sem_c, sem_d))

    def _start_gather(y_hbm, sidx_v, gbuf, blk, buf):
        for h in range(2):
            pltpu.async_copy(y_hbm.at[sidx_v.at[2 * blk + h]],
                             gbuf.at[buf, pl.ds(h * (KB // 2), KB // 2)],
                             sems[buf][h])

    def _wait_gather(y_hbm, sidx_v, gbuf, blk, buf):
        for h in range(2):
            pltpu.make_async_copy(y_hbm.at[sidx_v.at[2 * blk + h]],
                                  gbuf.at[buf, pl.ds(h * (KB // 2), KB // 2)],
                                  sems[buf][h]).wait()
    for i in range(3):
        chunk = c * 3 + i
        pltpu.sync_copy(y_hbm.at[pl.ds(chunk * NP + r0, RPT_A)], acc_sh.at[pl.ds(r0, RPT_A)])
        plsc.subcore_barrier()

        def body(jo, carry):
            pltpu.sync_copy(
                sidx_hbm.at[pl.ds(((chunk * 16 + s) * NBLK + jo * GB) * 2, 2 * GB)],
                sidx_v)
            pltpu.sync_copy(didx_hbm.at[pl.ds(s * NBLK + jo * GB, GB)], didx_v)
            _start_gather(y_hbm, sidx_v, gbuf, 0, 0)
            for g in range(GB):
                cur = g % 2
                _wait_gather(y_hbm, sidx_v, gbuf, g, cur)
                if g + 1 < GB:
                    _start_gather(y_hbm, sidx_v, gbuf, g + 1, 1 - cur)
                pltpu.sync_copy(gbuf.at[cur], acc_sh.at[didx_v.at[g]], add=True)
            return carry

        lax.fori_loop(0, NBLK // GB, body, 0)
        plsc.subcore_barrier()
        pltpu.sync_copy(acc_sh.at[pl.ds(r0, RPT_A)], out_hbm.at[pl.ds(chunk * NP + r0, RPT_A)])
        plsc.subcore_barrier()


def _mm_body(x_ref, czw_ref, lzw_ref, chw_ref, lhw_ref, deg_ref, out_ref):
    wz = jnp.dot(czw_ref[...], lzw_ref[...], preferred_element_type=jnp.float32)
    wh = jnp.dot(chw_ref[...], lhw_ref[...], preferred_element_type=jnp.float32)
    w = jnp.concatenate([wz, wh], axis=1)
    dinv = lax.rsqrt(deg_ref[0, :, 0:1] + deg_ref[1, :, 0:1])
    y0 = jnp.dot(x_ref[0], w, preferred_element_type=jnp.float32)
    y1 = jnp.dot(x_ref[1], w, preferred_element_type=jnp.float32)
    out_ref[0] = jnp.concatenate([y0, y1], axis=1) * dinv


_mm_call = pl.pallas_call(
    _mm_body,
    grid=(NCHUNK, NA // BN),
    in_specs=[
        pl.BlockSpec((2, BN, FI), lambda ci, i: (ci, i, 0)),
        pl.BlockSpec((FI, FO), lambda ci, i: (0, 0)),
        pl.BlockSpec((FO, FO), lambda ci, i: (0, 0)),
        pl.BlockSpec((FI, FO), lambda ci, i: (0, 0)),
        pl.BlockSpec((FO, FO), lambda ci, i: (0, 0)),
        pl.BlockSpec((2, BN, DW), lambda ci, i: (0, i, 0)),
    ],
    out_specs=pl.BlockSpec((1, BN, FI), lambda ci, i: (ci, i, 0)),
    out_shape=jax.ShapeDtypeStruct((NCHUNK, NP, FI), jnp.float32),
)


def _fin_body(agg_ref, deg_ref, att_ref, czb_ref, lzw_ref, lzb_ref, chb_ref,
              lhw_ref, lhb_ref, wout_ref, bout_ref, out_ref):
    probs = jax.nn.softmax(att_ref[...], axis=1)
    bz = jnp.dot(czb_ref[...], lzw_ref[...], preferred_element_type=jnp.float32) + lzb_ref[...]
    bh = jnp.dot(chb_ref[...], lhw_ref[...], preferred_element_type=jnp.float32) + lhb_ref[...]
    dinv = lax.rsqrt(deg_ref[0, :, 0:1] + deg_ref[1, :, 0:1])
    hacc = jnp.zeros((BN, FO), dtype=jnp.float32)
    for p in range(NPER):
        cch = p // 2
        off = 64 * (p % 2)
        a = agg_ref[cch, :, off:off + FO] * dinv + bz
        b = agg_ref[cch, :, off + FO:off + 2 * FO] * dinv + bh
        hp = (1.0 - jax.nn.sigmoid(a)) * jnp.tanh(b)
        hacc = hacc + probs[0, p] * hp
    out_ref[...] = (jnp.dot(jnp.maximum(hacc, 0.0), wout_ref[...],
                            preferred_element_type=jnp.float32) + bout_ref[...])


_fin_call = pl.pallas_call(
    _fin_body,
    grid=(NA // BN,),
    in_specs=[
        pl.BlockSpec((NCHUNK, BN, FI), lambda i: (0, i, 0)),
        pl.BlockSpec((2, BN, DW), lambda i: (0, i, 0)),
        pl.BlockSpec((1, NPER), lambda i: (0, 0)),
        pl.BlockSpec((1, FO), lambda i: (0, 0)),
        pl.BlockSpec((FO, FO), lambda i: (0, 0)),
        pl.BlockSpec((1, FO), lambda i: (0, 0)),
        pl.BlockSpec((1, FO), lambda i: (0, 0)),
        pl.BlockSpec((FO, FO), lambda i: (0, 0)),
        pl.BlockSpec((1, FO), lambda i: (0, 0)),
        pl.BlockSpec((FO, NPER), lambda i: (0, 0)),
        pl.BlockSpec((1, NPER), lambda i: (0, 0)),
    ],
    out_specs=pl.BlockSpec((BN, NPER), lambda i: (i, 0)),
    out_shape=jax.ShapeDtypeStruct((NA, NPER), jnp.float32),
)


def kernel(x, edge_index, attention, conv_z_W, conv_z_b, lin_z_W, lin_z_b,
           conv_r_W, conv_r_b, lin_r_W, lin_r_b, conv_h_W, conv_h_b,
           lin_h_W, lin_h_b, out_W, out_b):
    src = edge_index[0]
    dst = edge_index[1]

    pad = ECP - EDG
    ar = jnp.arange(pad, dtype=jnp.int32)
    src_p = jnp.concatenate([src, (ar * 97) % NA]).reshape(16 * NBLK, KB)
    dst_p = jnp.concatenate([dst, NA + (ar % (NP - NA))]).reshape(16 * NBLK, KB)
    sidx = (src_p[None] + (jnp.arange(NCHUNK, dtype=jnp.int32) * NP)[:, None, None])
    sidx = sidx.reshape(NCHUNK * 16 * NBLK * 2, KB // 2)

    ard = jnp.arange(EDP - EDG, dtype=jnp.int32)
    dstd = jnp.concatenate([dst, NA + (ard % (NP - NA))]).reshape(32 * NBLKA, KA)
    init = jnp.concatenate([jnp.ones((NP, DW), jnp.float32),
                            jnp.zeros((NP, DW), jnp.float32)])

    deg2 = _deg_kernel(dstd, init).reshape(2, NP, DW)
    x_t = jnp.transpose(x, (2, 0, 1))
    y = _mm_call(x_t, conv_z_W, lin_z_W[:FO], conv_h_W, lin_h_W[:FO], deg2)
    agg = _agg_kernel(y.reshape(NCHUNK * NP, FI), sidx, dst_p)
    out = _fin_call(agg.reshape(NCHUNK, NP, FI), deg2, attention.reshape(1, NPER),
                    conv_z_b.reshape(1, FO), lin_z_W[:FO], lin_z_b.reshape(1, FO),
                    conv_h_b.reshape(1, FO), lin_h_W[:FO], lin_h_b.reshape(1, FO),
                    out_W, out_b.reshape(1, NPER))
    return out
